# Initial kernel scaffold; baseline (speedup 1.0000x reference)
import functools

import numpy as np
import jax
import jax.numpy as jnp
from jax import lax
from jax.experimental import pallas as pl
from jax.experimental.pallas import tpu as pltpu
from jax.experimental.pallas import tpu_sc as plsc

KK = 12
MM = 8
NB = 4
NPTS = 1024
NPAIR = 66

_II, _JJ = np.triu_indices(KK, 1)
_GIT = np.zeros((KK, NPAIR), np.float32)
_GJT = np.zeros((KK, NPAIR), np.float32)
_GIT[_II, np.arange(NPAIR)] = 1.0
_GJT[_JJ, np.arange(NPAIR)] = 1.0

_NT = (((1,), (1,)), ((), ()))
_HI = lax.Precision.HIGHEST


def _dot(a, b):
    return lax.dot_general(a, b, _NT, precision=_HI,
                           preferred_element_type=jnp.float32)


def _gelu_t(t):
    return t * (lax.erf(t / np.sqrt(2)) + 1) / 2


def _knn(featT):
    C = featT.shape[2]

    def body(ft_ref, out_ref):
        b = pl.program_id(0)
        ft = ft_ref[0]
        xx = jnp.sum(ft * ft, axis=1, keepdims=True)
        fa = jnp.concatenate([ft, jnp.ones((NPTS, 1), jnp.float32)], axis=1)
        gb = jnp.concatenate([2.0 * ft, -xx], axis=1)
        pd = _dot(fa, gb)
        iota = lax.broadcasted_iota(jnp.int32, (NPTS, NPTS), 1)
        cols = []
        for _ in range(KK):
            mx = jnp.max(pd, axis=1, keepdims=True)
            sel = jnp.min(jnp.where(pd == mx, iota, NPTS), axis=1,
                          keepdims=True)
            cols.append(sel + b * NPTS)
            pd = jnp.where(iota == sel, -jnp.inf, pd)
        out_ref[0] = jnp.concatenate(cols, axis=1)

    return pl.pallas_call(
        body,
        grid=(NB,),
        in_specs=[pl.BlockSpec((1, NPTS, C), lambda b: (b, 0, 0))],
        out_specs=pl.BlockSpec((1, NPTS, KK), lambda b: (b, 0, 0)),
        out_shape=jax.ShapeDtypeStruct((NB, NPTS, KK), jnp.int32),
    )(featT)


_SC_NC = 2
_SC_NS = 16
_SC_NW = _SC_NC * _SC_NS


def _sc_gather(coords, gidx):
    tot = gidx.shape[0]
    ch = tot // _SC_NW
    nrow = coords.shape[0]
    mesh = plsc.VectorSubcoreMesh(core_axis_name="c", subcore_axis_name="s")

    @functools.partial(
        pl.kernel,
        mesh=mesh,
        out_type=jax.ShapeDtypeStruct((3, tot), jnp.float32),
        scratch_types=[
            pltpu.VMEM((nrow, 3), jnp.float32),
            pltpu.VMEM((ch,), jnp.int32),
            pltpu.VMEM((3, ch), jnp.float32),
        ],
    )
    def k(coords_hbm, gidx_hbm, out_hbm, tab_v, idx_v, out_v):
        wid = lax.axis_index("s") * _SC_NC + lax.axis_index("c")
        base = wid * ch
        pltpu.sync_copy(coords_hbm, tab_v)
        pltpu.sync_copy(gidx_hbm.at[pl.ds(base, ch)], idx_v)

        def step(j, carry):
            iv = idx_v[pl.ds(j * 16, 16)]
            for c in range(3):
                cc = jnp.full((16,), c, jnp.int32)
                out_v[c, pl.ds(j * 16, 16)] = plsc.load_gather(tab_v, [iv, cc])
            return carry

        lax.fori_loop(0, ch // 16, step, 0)
        for c in range(3):
            pltpu.sync_copy(out_v.at[c], out_hbm.at[c, pl.ds(base, ch)])

    return k(coords, gidx)


def _normals(pts, nbr):
    P = 128

    def body(gi_ref, gj_ref, pts_ref, nbr_ref, out_ref):
        ctr = pts_ref[0]
        giT = gi_ref[...]
        gjT = gj_ref[...]
        vx = nbr_ref[0, 0] - ctr[:, 0:1]
        vy = nbr_ref[1, 0] - ctr[:, 1:2]
        vz = nbr_ref[2, 0] - ctr[:, 2:3]
        ax_ = _dot(vx, giT); ay_ = _dot(vy, giT); az_ = _dot(vz, giT)
        bx_ = _dot(vx, gjT); by_ = _dot(vy, gjT); bz_ = _dot(vz, gjT)
        nx = ay_ * bz_ - az_ * by_
        ny = az_ * bx_ - ax_ * bz_
        nz = ax_ * by_ - ay_ * bx_
        nmag = jnp.sqrt(nx * nx + ny * ny + nz * nz + 1e-24)
        valid = nmag > 1e-6
        zero = jnp.zeros_like(nx)
        ux = jnp.where(valid, nx / nmag, zero)
        uy = jnp.where(valid, ny / nmag, zero)
        uz = jnp.where(valid, nz / nmag, zero)
        d3 = (ux[:, :, None] * ux[:, None, :]
              + uy[:, :, None] * uy[:, None, :]
              + uz[:, :, None] * uz[:, None, :])
        sim = jnp.where(valid, jnp.sum(jnp.abs(d3), axis=2), -jnp.inf)
        iota = lax.broadcasted_iota(jnp.int32, (P, NPAIR), 1)
        sx = jnp.zeros((P, 1), jnp.float32)
        sy = jnp.zeros((P, 1), jnp.float32)
        sz = jnp.zeros((P, 1), jnp.float32)
        for _ in range(MM):
            mx = jnp.max(sim, axis=1, keepdims=True)
            sel = jnp.min(jnp.where(sim == mx, iota, NPAIR), axis=1,
                          keepdims=True)
            msk = iota == sel
            sx = sx + jnp.sum(jnp.where(msk, ux, zero), axis=1, keepdims=True)
            sy = sy + jnp.sum(jnp.where(msk, uy, zero), axis=1, keepdims=True)
            sz = sz + jnp.sum(jnp.where(msk, uz, zero), axis=1, keepdims=True)
            sim = jnp.where(msk, -jnp.inf, sim)
        sx = sx / MM; sy = sy / MM; sz = sz / MM
        den = jnp.sqrt(sx * sx + sy * sy + sz * sz + 1e-24) + 1e-6
        out_ref[0] = jnp.concatenate([sx / den, sy / den, sz / den], axis=1)

    return pl.pallas_call(
        body,
        grid=(NB, NPTS // P),
        in_specs=[
            pl.BlockSpec((KK, NPAIR), lambda b, p: (0, 0)),
            pl.BlockSpec((KK, NPAIR), lambda b, p: (0, 0)),
            pl.BlockSpec((1, P, 3), lambda b, p: (b, p, 0)),
            pl.BlockSpec((3, 1, P, KK), lambda b, p: (0, b, p, 0)),
        ],
        out_specs=pl.BlockSpec((1, P, 3), lambda b, p: (b, p, 0)),
        out_shape=jax.ShapeDtypeStruct((NB, NPTS, 3), jnp.float32),
    )(jnp.asarray(_GIT), jnp.asarray(_GJT), pts, nbr)


def _vec_avg_T(featT):
    gidx = _knn(featT)
    pts = featT[:, :, :3]
    coords = pts.reshape(NB * NPTS, 3)
    nbr = _sc_gather(coords, gidx.reshape(-1)).reshape(3, NB, NPTS, KK)
    return _normals(pts, nbr)


def _conv_bn_gelu(parts, W, g, b):
    nparts = len(parts)
    cout = W.shape[0]

    def body(*refs):
        part_refs = refs[:nparts]
        w_ref, g_ref, b_ref, out_ref = refs[nparts:]
        hs = []
        s1 = jnp.zeros((1, cout), jnp.float32)
        for bb in range(NB):
            if nparts > 1:
                xc = jnp.concatenate([r[bb] for r in part_refs], axis=1)
            else:
                xc = part_refs[0][bb]
            h = _dot(xc, w_ref[...])
            hs.append(h)
            s1 = s1 + jnp.sum(h, axis=0, keepdims=True)
        mu = s1 / (NB * NPTS)
        s2 = jnp.zeros((1, cout), jnp.float32)
        for h in hs:
            s2 = s2 + jnp.sum((h - mu) ** 2, axis=0, keepdims=True)
        denom = jnp.sqrt(s2 / (NB * NPTS) + 1e-5)
        for bb in range(NB):
            t = g_ref[...][None, :] * (hs[bb] - mu) / denom + b_ref[...][None, :]
            out_ref[bb] = _gelu_t(t)

    return pl.pallas_call(
        body,
        out_shape=jax.ShapeDtypeStruct((NB, NPTS, cout), jnp.float32),
    )(*parts, W, g, b)


def _conv4pool(x1T, x2T, x3T, W4, g4, b4):
    CB = 128
    nblk = W4.shape[0] // CB

    def body(x1_ref, x2_ref, x3_ref, w_ref, g_ref, b_ref, out_ref):
        hs = []
        s1 = jnp.zeros((1, CB), jnp.float32)
        for bb in range(NB):
            xc = jnp.concatenate([x1_ref[bb], x2_ref[bb], x3_ref[bb]], axis=1)
            h = _dot(xc, w_ref[...])
            hs.append(h)
            s1 = s1 + jnp.sum(h, axis=0, keepdims=True)
        mu = s1 / (NB * NPTS)
        s2 = jnp.zeros((1, CB), jnp.float32)
        for h in hs:
            s2 = s2 + jnp.sum((h - mu) ** 2, axis=0, keepdims=True)
        denom = jnp.sqrt(s2 / (NB * NPTS) + 1e-5)
        rows = []
        for bb in range(NB):
            t = g_ref[0][None, :] * (hs[bb] - mu) / denom + b_ref[0][None, :]
            rows.append(jnp.max(_gelu_t(t), axis=0, keepdims=True))
        out_ref[...] = jnp.concatenate(rows, axis=0)

    return pl.pallas_call(
        body,
        grid=(nblk,),
        in_specs=[
            pl.BlockSpec((NB, NPTS, 64), lambda c: (0, 0, 0)),
            pl.BlockSpec((NB, NPTS, 64), lambda c: (0, 0, 0)),
            pl.BlockSpec((NB, NPTS, 128), lambda c: (0, 0, 0)),
            pl.BlockSpec((CB, 256), lambda c: (c, 0)),
            pl.BlockSpec((1, CB), lambda c: (c, 0)),
            pl.BlockSpec((1, CB), lambda c: (c, 0)),
        ],
        out_specs=pl.BlockSpec((NB, CB), lambda c: (0, c)),
        out_shape=jax.ShapeDtypeStruct((NB, W4.shape[0]), jnp.float32),
    )(x1T, x2T, x3T, W4, g4.reshape(nblk, CB), b4.reshape(nblk, CB))


def _head(pooled, L1, g5, b5, L2, bl2):
    def body(p_ref, l1_ref, g_ref, b_ref, l2_ref, bl_ref, out_ref):
        z = _dot(p_ref[...], l1_ref[...])
        mu = jnp.mean(z, axis=0, keepdims=True)
        var = jnp.mean((z - mu) ** 2, axis=0, keepdims=True)
        z = g_ref[...][None, :] * (z - mu) / jnp.sqrt(var + 1e-5) \
            + b_ref[...][None, :]
        z = _gelu_t(z)
        out_ref[...] = _dot(z, l2_ref[...]) + bl_ref[...][None, :]

    return pl.pallas_call(
        body,
        out_shape=jax.ShapeDtypeStruct((NB, L2.shape[0]), jnp.float32),
    )(pooled, L1, g5, b5, L2, bl2)


def kernel(x, W1, g1, b1, W2, g2, b2, W3, g3, b3, W4, g4, b4,
           L1, g5, b5, L2, bl2):
    xT = jnp.transpose(x, (0, 2, 1))
    n1 = _vec_avg_T(xT)
    x1T = _conv_bn_gelu([xT, n1], W1, g1, b1)
    n2 = _vec_avg_T(x1T)
    x2T = _conv_bn_gelu([x1T, n2], W2, g2, b2)
    n3 = _vec_avg_T(x2T)
    x3T = _conv_bn_gelu([x2T, n3], W3, g3, b3)
    pooled = _conv4pool(x1T, x2T, x3T, W4, g4, b4)
    return _head(pooled, L1, g5, b5, L2, bl2)


# speedup vs baseline: 4.4205x; 4.4205x over previous
import functools

import numpy as np
import jax
import jax.numpy as jnp
from jax import lax
from jax.experimental import pallas as pl
from jax.experimental.pallas import tpu as pltpu
from jax.experimental.pallas import tpu_sc as plsc

KK = 12
MM = 8
NB = 4
NPTS = 1024
NPAIR = 66

_II, _JJ = np.triu_indices(KK, 1)
_GIT = np.zeros((KK, NPAIR), np.float32)
_GJT = np.zeros((KK, NPAIR), np.float32)
_GIT[_II, np.arange(NPAIR)] = 1.0
_GJT[_JJ, np.arange(NPAIR)] = 1.0

_NT = (((1,), (1,)), ((), ()))
_HI = lax.Precision.HIGHEST


def _dot(a, b):
    return lax.dot_general(a, b, _NT, precision=_HI,
                           preferred_element_type=jnp.float32)


def _dotn(a, b):
    return lax.dot_general(a, b, (((1,), (0,)), ((), ())), precision=_HI,
                           preferred_element_type=jnp.float32)


def _dot_def(a, b):
    return lax.dot_general(a, b, _NT, preferred_element_type=jnp.float32)


def _gelu_t(t):
    return t * (lax.erf(t / np.sqrt(2)) + 1) / 2


def _knn(featT):
    C = featT.shape[2]

    def body(ft_ref, out_ref):
        b = pl.program_id(0)
        ft = ft_ref[0]
        xx = jnp.sum(ft * ft, axis=1, keepdims=True)
        inner = -2.0 * _dot_def(ft, ft)
        pd = -xx - inner - jnp.transpose(xx, (1, 0))
        iota = lax.broadcasted_iota(jnp.int32, (NPTS, NPTS), 1)
        cols = []
        for _ in range(KK):
            mx = jnp.max(pd, axis=1, keepdims=True)
            sel = jnp.min(jnp.where(pd == mx, iota, NPTS), axis=1,
                          keepdims=True)
            cols.append(sel + b * NPTS)
            pd = jnp.where(iota == sel, -jnp.inf, pd)
        out_ref[0] = jnp.concatenate(cols, axis=1)

    return pl.pallas_call(
        body,
        grid=(NB,),
        in_specs=[pl.BlockSpec((1, NPTS, C), lambda b: (b, 0, 0))],
        out_specs=pl.BlockSpec((1, NPTS, KK), lambda b: (b, 0, 0)),
        out_shape=jax.ShapeDtypeStruct((NB, NPTS, KK), jnp.int32),
    )(featT)


_SC_NC = 2
_SC_NS = 16
_SC_NW = _SC_NC * _SC_NS


def _sc_gather(coords, gidx):
    tot = gidx.shape[0]
    ch = tot // _SC_NW
    nrow = coords.shape[0] // 3
    mesh = plsc.VectorSubcoreMesh(core_axis_name="c", subcore_axis_name="s")

    @functools.partial(
        pl.kernel,
        mesh=mesh,
        compiler_params=pltpu.CompilerParams(needs_layout_passes=False),
        out_type=jax.ShapeDtypeStruct((3 * tot,), jnp.float32),
        scratch_types=[
            pltpu.VMEM((nrow,), jnp.float32),
            pltpu.VMEM((nrow,), jnp.float32),
            pltpu.VMEM((nrow,), jnp.float32),
            pltpu.VMEM((ch,), jnp.int32),
            pltpu.VMEM((ch,), jnp.float32),
            pltpu.VMEM((ch,), jnp.float32),
            pltpu.VMEM((ch,), jnp.float32),
        ],
    )
    def k(coords_hbm, gidx_hbm, out_hbm, t0_v, t1_v, t2_v,
          idx_v, o0_v, o1_v, o2_v):
        wid = lax.axis_index("s") * _SC_NC + lax.axis_index("c")
        base = wid * ch
        tabs = (t0_v, t1_v, t2_v)
        outs = (o0_v, o1_v, o2_v)
        for c in range(3):
            pltpu.sync_copy(coords_hbm.at[pl.ds(c * nrow, nrow)], tabs[c])
        pltpu.sync_copy(gidx_hbm.at[pl.ds(base, ch)], idx_v)

        def step(j, carry):
            iv = idx_v[pl.ds(j * 16, 16)]
            for c in range(3):
                outs[c][pl.ds(j * 16, 16)] = plsc.load_gather(tabs[c], [iv])
            return carry

        lax.fori_loop(0, ch // 16, step, 0)
        for c in range(3):
            pltpu.sync_copy(outs[c], out_hbm.at[pl.ds(c * tot + base, ch)])

    return k(coords, gidx)


def _normals(pts, nbr):
    P = 128

    def body(gi_ref, gj_ref, pts_ref, nbr_ref, out_ref):
        ctr = pts_ref[0]
        giT = gi_ref[...]
        gjT = gj_ref[...]
        vx = nbr_ref[0, 0] - ctr[:, 0:1]
        vy = nbr_ref[1, 0] - ctr[:, 1:2]
        vz = nbr_ref[2, 0] - ctr[:, 2:3]
        ax_ = _dotn(vx, giT); ay_ = _dotn(vy, giT); az_ = _dotn(vz, giT)
        bx_ = _dotn(vx, gjT); by_ = _dotn(vy, gjT); bz_ = _dotn(vz, gjT)
        nx = ay_ * bz_ - az_ * by_
        ny = az_ * bx_ - ax_ * bz_
        nz = ax_ * by_ - ay_ * bx_
        nmag = jnp.sqrt(nx * nx + ny * ny + nz * nz + 1e-24)
        valid = nmag > 1e-6
        zero = jnp.zeros_like(nx)
        ux = jnp.where(valid, nx / nmag, zero)
        uy = jnp.where(valid, ny / nmag, zero)
        uz = jnp.where(valid, nz / nmag, zero)
        qx = ux.astype(jnp.bfloat16).astype(jnp.float32)
        qy = uy.astype(jnp.bfloat16).astype(jnp.float32)
        qz = uz.astype(jnp.bfloat16).astype(jnp.float32)
        d3 = (qx[:, :, None] * qx[:, None, :]
              + qy[:, :, None] * qy[:, None, :]
              + qz[:, :, None] * qz[:, None, :])
        sim = jnp.where(valid, jnp.sum(jnp.abs(d3), axis=2), -jnp.inf)
        iota = lax.broadcasted_iota(jnp.int32, (P, NPAIR), 1)
        sx = jnp.zeros((P, 1), jnp.float32)
        sy = jnp.zeros((P, 1), jnp.float32)
        sz = jnp.zeros((P, 1), jnp.float32)
        for _ in range(MM):
            mx = jnp.max(sim, axis=1, keepdims=True)
            sel = jnp.min(jnp.where(sim == mx, iota, NPAIR), axis=1,
                          keepdims=True)
            msk = iota == sel
            sx = sx + jnp.sum(jnp.where(msk, ux, zero), axis=1, keepdims=True)
            sy = sy + jnp.sum(jnp.where(msk, uy, zero), axis=1, keepdims=True)
            sz = sz + jnp.sum(jnp.where(msk, uz, zero), axis=1, keepdims=True)
            sim = jnp.where(msk, -jnp.inf, sim)
        sx = sx / MM; sy = sy / MM; sz = sz / MM
        den = jnp.sqrt(sx * sx + sy * sy + sz * sz + 1e-24) + 1e-6
        out_ref[0] = jnp.concatenate([sx / den, sy / den, sz / den], axis=1)

    return pl.pallas_call(
        body,
        grid=(NB, NPTS // P),
        in_specs=[
            pl.BlockSpec((KK, NPAIR), lambda b, p: (0, 0)),
            pl.BlockSpec((KK, NPAIR), lambda b, p: (0, 0)),
            pl.BlockSpec((1, P, 3), lambda b, p: (b, p, 0)),
            pl.BlockSpec((3, 1, P, KK), lambda b, p: (0, b, p, 0)),
        ],
        out_specs=pl.BlockSpec((1, P, 3), lambda b, p: (b, p, 0)),
        out_shape=jax.ShapeDtypeStruct((NB, NPTS, 3), jnp.float32),
    )(jnp.asarray(_GIT), jnp.asarray(_GJT), pts, nbr)


def _vec_avg_T(featT):
    gidx = _knn(featT)
    pts = featT[:, :, :3]
    coords = jnp.transpose(pts, (2, 0, 1)).reshape(-1)
    nbr = _sc_gather(coords, gidx.reshape(-1)).reshape(3, NB, NPTS, KK)
    return _normals(pts, nbr)


def _conv_bn_gelu(parts, W, g, b):
    nparts = len(parts)
    cout = W.shape[0]

    def body(*refs):
        part_refs = refs[:nparts]
        w_ref, g_ref, b_ref, out_ref = refs[nparts:]
        hs = []
        s1 = jnp.zeros((1, cout), jnp.float32)
        for bb in range(NB):
            if nparts > 1:
                xc = jnp.concatenate([r[bb] for r in part_refs], axis=1)
            else:
                xc = part_refs[0][bb]
            h = _dot_def(xc, w_ref[...])
            hs.append(h)
            s1 = s1 + jnp.sum(h, axis=0, keepdims=True)
        mu = s1 / (NB * NPTS)
        s2 = jnp.zeros((1, cout), jnp.float32)
        for h in hs:
            s2 = s2 + jnp.sum((h - mu) ** 2, axis=0, keepdims=True)
        denom = jnp.sqrt(s2 / (NB * NPTS) + 1e-5)
        for bb in range(NB):
            t = g_ref[...][None, :] * (hs[bb] - mu) / denom + b_ref[...][None, :]
            out_ref[bb] = _gelu_t(t)

    return pl.pallas_call(
        body,
        out_shape=jax.ShapeDtypeStruct((NB, NPTS, cout), jnp.float32),
    )(*parts, W, g, b)


def _conv4pool(x1T, x2T, x3T, W4, g4, b4):
    CB = 128
    nblk = W4.shape[0] // CB

    def body(x1_ref, x2_ref, x3_ref, w_ref, g_ref, b_ref, out_ref):
        hs = []
        s1 = jnp.zeros((1, CB), jnp.float32)
        for bb in range(NB):
            xc = jnp.concatenate([x1_ref[bb], x2_ref[bb], x3_ref[bb]], axis=1)
            h = _dot_def(xc, w_ref[...])
            hs.append(h)
            s1 = s1 + jnp.sum(h, axis=0, keepdims=True)
        mu = s1 / (NB * NPTS)
        s2 = jnp.zeros((1, CB), jnp.float32)
        for h in hs:
            s2 = s2 + jnp.sum((h - mu) ** 2, axis=0, keepdims=True)
        denom = jnp.sqrt(s2 / (NB * NPTS) + 1e-5)
        rows = []
        for bb in range(NB):
            t = g_ref[0, 0][None, :] * (hs[bb] - mu) / denom \
                + b_ref[0, 0][None, :]
            rows.append(jnp.max(_gelu_t(t), axis=0, keepdims=True))
        out_ref[...] = jnp.concatenate(rows, axis=0)

    return pl.pallas_call(
        body,
        grid=(nblk,),
        in_specs=[
            pl.BlockSpec((NB, NPTS, 64), lambda c: (0, 0, 0)),
            pl.BlockSpec((NB, NPTS, 64), lambda c: (0, 0, 0)),
            pl.BlockSpec((NB, NPTS, 128), lambda c: (0, 0, 0)),
            pl.BlockSpec((CB, 256), lambda c: (c, 0)),
            pl.BlockSpec((1, 1, CB), lambda c: (c, 0, 0)),
            pl.BlockSpec((1, 1, CB), lambda c: (c, 0, 0)),
        ],
        out_specs=pl.BlockSpec((NB, CB), lambda c: (0, c)),
        out_shape=jax.ShapeDtypeStruct((NB, W4.shape[0]), jnp.float32),
    )(x1T, x2T, x3T, W4, g4.reshape(nblk, 1, CB), b4.reshape(nblk, 1, CB))


def _head(pooled, L1, g5, b5, L2, bl2):
    def body(p_ref, l1_ref, g_ref, b_ref, l2_ref, bl_ref, out_ref):
        z = _dot_def(p_ref[...], l1_ref[...])
        mu = jnp.mean(z, axis=0, keepdims=True)
        var = jnp.mean((z - mu) ** 2, axis=0, keepdims=True)
        z = g_ref[...][None, :] * (z - mu) / jnp.sqrt(var + 1e-5) \
            + b_ref[...][None, :]
        z = _gelu_t(z)
        out_ref[...] = _dot_def(z, l2_ref[...]) + bl_ref[...][None, :]

    return pl.pallas_call(
        body,
        out_shape=jax.ShapeDtypeStruct((NB, L2.shape[0]), jnp.float32),
    )(pooled, L1, g5, b5, L2, bl2)


def kernel(x, W1, g1, b1, W2, g2, b2, W3, g3, b3, W4, g4, b4,
           L1, g5, b5, L2, bl2):
    xT = jnp.transpose(x, (0, 2, 1))
    n1 = _vec_avg_T(xT)
    x1T = _conv_bn_gelu([xT, n1], W1, g1, b1)
    n2 = _vec_avg_T(x1T)
    x2T = _conv_bn_gelu([x1T, n2], W2, g2, b2)
    n3 = _vec_avg_T(x2T)
    x3T = _conv_bn_gelu([x2T, n3], W3, g3, b3)
    pooled = _conv4pool(x1T, x2T, x3T, W4, g4, b4)
    return _head(pooled, L1, g5, b5, L2, bl2)
